# Initial kernel scaffold; baseline (speedup 1.0000x reference)
import jax
import jax.numpy as jnp
from jax.experimental import pallas as pl


def kernel(inputs, user_memory, item_memory, uW_ih, uW_hh, ub_ih, ub_hh, iW_ih, iW_hh, ib_ih, ib_hh):
    raise NotImplementedError("write your pallas kernel here")


# speedup vs baseline: 4.2250x; 4.2250x over previous
import functools

import jax
import jax.numpy as jnp
from jax import lax
from jax.experimental import pallas as pl
from jax.experimental.pallas import tpu as pltpu
from jax.experimental.pallas import tpu_sc as plsc

EMBED = 32
UF = 16
IF = 16

_NC = 2
_NS = 16


def _sc_gather(uflat, iflat, idx_u, idx_i):
    T = idx_u.shape[0]
    NW = _NC * _NS
    TPW = T // NW
    mesh = plsc.VectorSubcoreMesh(core_axis_name="c", subcore_axis_name="s")

    @functools.partial(
        pl.kernel,
        mesh=mesh,
        out_type=(
            jax.ShapeDtypeStruct((T,), jnp.float32),
            jax.ShapeDtypeStruct((T,), jnp.float32),
        ),
        scratch_types=[
            pltpu.VMEM((TPW,), jnp.int32),
            pltpu.VMEM((TPW,), jnp.float32),
            pltpu.VMEM((TPW,), jnp.int32),
            pltpu.VMEM((TPW,), jnp.float32),
            pltpu.SemaphoreType.DMA,
            pltpu.SemaphoreType.DMA,
        ],
    )
    def gather_k(utab, itab, idxu_hbm, idxi_hbm, um_out, im_out,
                 idxu_v, rowsu_v, idxi_v, rowsi_v, semu, semi):
        wid = lax.axis_index("s") * _NC + lax.axis_index("c")
        base = wid * TPW
        pltpu.sync_copy(idxu_hbm.at[pl.ds(base, TPW)], idxu_v)
        pltpu.sync_copy(idxi_hbm.at[pl.ds(base, TPW)], idxi_v)
        cu = pltpu.async_copy(utab.at[idxu_v], rowsu_v, semu)
        ci = pltpu.async_copy(itab.at[idxi_v], rowsi_v, semi)
        cu.wait()
        ci.wait()
        pltpu.sync_copy(rowsu_v, um_out.at[pl.ds(base, TPW)])
        pltpu.sync_copy(rowsi_v, im_out.at[pl.ds(base, TPW)])

    return gather_k(uflat, iflat, idx_u, idx_i)


def _gru_body(inpT_ref, umT_ref, imT_ref, uW_ref, ubih_ref, ubhh_ref,
              iW_ref, ibih_ref, ibhh_ref, newuT_ref, newiT_ref, outT_ref):
    inpT = inpT_ref[...]
    umT = umT_ref[...]
    imT = imT_ref[...]
    ufT = inpT[2:2 + UF, :]
    itfT = inpT[2 + UF:2 + UF + IF, :]
    xuT = jnp.concatenate([umT, ufT, imT, itfT], axis=0)
    xiT = jnp.concatenate([imT, itfT, umT, ufT], axis=0)

    def cell(xT, W, bih, bhh):
        g = lax.dot_general(W, xT, (((1,), (0,)), ((), ())),
                            preferred_element_type=jnp.float32)
        g = g + bih
        r = jax.nn.sigmoid(g[:EMBED, :] + bhh[:EMBED, :])
        z = jax.nn.sigmoid(g[EMBED:2 * EMBED, :] + bhh[EMBED:2 * EMBED, :])
        n = jnp.tanh(g[2 * EMBED:, :] + r * bhh[2 * EMBED:, :])
        h = (1.0 - z) * n
        norm = jnp.maximum(jnp.sqrt(jnp.sum(h * h, axis=0, keepdims=True)), 1e-12)
        return h / norm

    nuT = cell(xuT, uW_ref[...], ubih_ref[...], ubhh_ref[...])
    niT = cell(xiT, iW_ref[...], ibih_ref[...], ibhh_ref[...])
    newuT_ref[...] = nuT
    newiT_ref[...] = niT
    outT_ref[...] = jnp.concatenate([inpT[:2, :], nuT, niT], axis=0)


def _gru(inputsT, umT, imT, uW_ih, ub_ih, ub_hh, iW_ih, ib_ih, ib_hh):
    B = inputsT.shape[1]
    f32 = jnp.float32
    return pl.pallas_call(
        _gru_body,
        out_shape=(
            jax.ShapeDtypeStruct((EMBED, B), f32),
            jax.ShapeDtypeStruct((EMBED, B), f32),
            jax.ShapeDtypeStruct((2 + 2 * EMBED, B), f32),
        ),
    )(inputsT, umT, imT,
      uW_ih, ub_ih.reshape(-1, 1), ub_hh.reshape(-1, 1),
      iW_ih, ib_ih.reshape(-1, 1), ib_hh.reshape(-1, 1))


_UBLK = 8


def _select_body(uids_ref, iids_ref, newuT_ref, newiT_ref, pu_ref, pi_ref,
                 uout_ref, iout_ref):
    i = pl.program_id(0)
    u0 = i * _UBLK
    urow = lax.broadcasted_iota(jnp.int32, (_UBLK, 1, 1), 0) + u0
    umask = urow == uids_ref[...].reshape(1, 1, -1)
    imask = urow == iids_ref[...].reshape(1, 1, -1)
    uout_ref[...] = jnp.where(umask, newuT_ref[...][None], pu_ref[...])
    iout_ref[...] = jnp.where(imask, newiT_ref[...][None], pi_ref[...])


def _select_scatter(uids, iids, newuT, newiT, pu, pi):
    N, E, B = pu.shape
    grid = (N // _UBLK,)
    const2 = lambda i: (0, 0)
    const3 = lambda i: (i, 0, 0)
    bank_spec = pl.BlockSpec((_UBLK, E, B), const3)
    return pl.pallas_call(
        _select_body,
        grid=grid,
        in_specs=[
            pl.BlockSpec((1, B), const2),
            pl.BlockSpec((1, B), const2),
            pl.BlockSpec((E, B), const2),
            pl.BlockSpec((E, B), const2),
            bank_spec,
            bank_spec,
        ],
        out_specs=(bank_spec, bank_spec),
        out_shape=(jax.ShapeDtypeStruct((N, E, B), pu.dtype),
                   jax.ShapeDtypeStruct((N, E, B), pi.dtype)),
    )(uids.reshape(1, B), iids.reshape(1, B), newuT, newiT, pu, pi)


def kernel(inputs, user_memory, item_memory, uW_ih, uW_hh, ub_ih, ub_hh,
           iW_ih, iW_hh, ib_ih, ib_hh):
    B = inputs.shape[0]
    NU = user_memory.shape[1]
    NI = item_memory.shape[1]
    uid = inputs[:, 0].astype(jnp.int32)
    iid = inputs[:, 1].astype(jnp.int32)
    ar = jnp.arange(B, dtype=jnp.int32)
    e = jnp.arange(EMBED, dtype=jnp.int32)
    pu = jnp.transpose(user_memory, (1, 2, 0))
    pi = jnp.transpose(item_memory, (1, 2, 0))
    idx_u = ((uid[:, None] * EMBED + e[None, :]) * B + ar[:, None]).reshape(-1)
    idx_i = ((iid[:, None] * EMBED + e[None, :]) * B + ar[:, None]).reshape(-1)

    um_flat, im_flat = _sc_gather(pu.reshape(-1), pi.reshape(-1), idx_u, idx_i)
    umT = um_flat.reshape(B, EMBED).T
    imT = im_flat.reshape(B, EMBED).T
    newuT, newiT, outT = _gru(inputs.T, umT, imT, uW_ih, ub_ih, ub_hh,
                              iW_ih, ib_ih, ib_hh)
    pu_new, pi_new = _select_scatter(uid, iid, newuT, newiT, pu, pi)
    new_user_memory = jnp.transpose(pu_new, (2, 0, 1))
    new_item_memory = jnp.transpose(pi_new, (2, 0, 1))
    return (outT.T, new_user_memory, new_item_memory)


# speedup vs baseline: 7.5908x; 1.7966x over previous
import jax
import jax.numpy as jnp
from jax import lax
from jax.experimental import pallas as pl
from jax.experimental.pallas import tpu as pltpu

EMBED = 32
UF = 16
IF = 16


_LANES = 128


def _gather_gru_body(uid_ref, iid_ref, inp_ref, uW_ref, ubih_ref, ubhh_ref,
                     iW_ref, ibih_ref, ibhh_ref, pu_ref, pi_ref,
                     newuT_ref, newiT_ref, out_ref,
                     umscr, imscr, semu, semi):
    B = inp_ref.shape[0]

    def issue(b, c):
        u = uid_ref[b]
        it = iid_ref[b]
        lt = pl.multiple_of((b // _LANES) * _LANES, _LANES)
        pltpu.make_async_copy(pu_ref.at[pl.ds(u, 1), :, pl.ds(lt, _LANES)],
                              umscr.at[pl.ds(b, 1)], semu).start()
        pltpu.make_async_copy(pi_ref.at[pl.ds(it, 1), :, pl.ds(lt, _LANES)],
                              imscr.at[pl.ds(b, 1)], semi).start()
        return c

    lax.fori_loop(0, B, issue, 0)
    du = pltpu.make_async_copy(pu_ref.at[pl.ds(0, 1), :, pl.ds(0, _LANES)],
                               umscr.at[pl.ds(0, 1)], semu)
    di = pltpu.make_async_copy(pi_ref.at[pl.ds(0, 1), :, pl.ds(0, _LANES)],
                               imscr.at[pl.ds(0, 1)], semi)

    def drain(b, c):
        du.wait()
        di.wait()
        return c

    lax.fori_loop(0, B, drain, 0)

    def extract(scr):
        s4 = scr[...].reshape(B // _LANES, _LANES, EMBED, _LANES)
        sel = lax.broadcasted_iota(jnp.int32, s4.shape, 1)
        lane = lax.broadcasted_iota(jnp.int32, s4.shape, 3)
        return jnp.sum(jnp.where(sel == lane, s4, 0.0), axis=3).reshape(B, EMBED)

    um = extract(umscr)
    im = extract(imscr)
    inp = inp_ref[...]
    uf = inp[:, 2:2 + UF]
    itf = inp[:, 2 + UF:2 + UF + IF]
    xu = jnp.concatenate([um, uf, im, itf], axis=1)
    xi = jnp.concatenate([im, itf, um, uf], axis=1)

    def cell(x, W, bih, bhh):
        g = lax.dot_general(x, W, (((1,), (1,)), ((), ())),
                            preferred_element_type=jnp.float32)
        g = g + bih
        r = jax.nn.sigmoid(g[:, :EMBED] + bhh[:, :EMBED])
        z = jax.nn.sigmoid(g[:, EMBED:2 * EMBED] + bhh[:, EMBED:2 * EMBED])
        n = jnp.tanh(g[:, 2 * EMBED:] + r * bhh[:, 2 * EMBED:])
        h = (1.0 - z) * n
        norm = jnp.maximum(jnp.sqrt(jnp.sum(h * h, axis=1, keepdims=True)), 1e-12)
        return h / norm

    nu = cell(xu, uW_ref[...], ubih_ref[...], ubhh_ref[...])
    ni = cell(xi, iW_ref[...], ibih_ref[...], ibhh_ref[...])
    newuT_ref[...] = nu.T
    newiT_ref[...] = ni.T
    out_ref[...] = jnp.concatenate([inp[:, :2], nu, ni], axis=1)


def _gather_gru(uid, iid, inputs, uW_ih, ub_ih, ub_hh, iW_ih, ib_ih, ib_hh,
                pu, pi):
    B = inputs.shape[0]
    f32 = jnp.float32
    smem = pl.BlockSpec(memory_space=pltpu.MemorySpace.SMEM)
    hbm = pl.BlockSpec(memory_space=pltpu.MemorySpace.HBM)
    vmem = pl.BlockSpec(memory_space=pltpu.MemorySpace.VMEM)
    return pl.pallas_call(
        _gather_gru_body,
        in_specs=[smem, smem, vmem, vmem, vmem, vmem, vmem, vmem, vmem,
                  hbm, hbm],
        out_shape=(
            jax.ShapeDtypeStruct((EMBED, B), f32),
            jax.ShapeDtypeStruct((EMBED, B), f32),
            jax.ShapeDtypeStruct((B, 2 + 2 * EMBED), f32),
        ),
        scratch_shapes=[
            pltpu.VMEM((B, EMBED, _LANES), f32),
            pltpu.VMEM((B, EMBED, _LANES), f32),
            pltpu.SemaphoreType.DMA,
            pltpu.SemaphoreType.DMA,
        ],
    )(uid, iid, inputs,
      uW_ih, ub_ih.reshape(1, -1), ub_hh.reshape(1, -1),
      iW_ih, ib_ih.reshape(1, -1), ib_hh.reshape(1, -1),
      pu, pi)


_UBLK = 8


def _select_body(uids_ref, iids_ref, newuT_ref, newiT_ref, pu_ref, pi_ref,
                 uout_ref, iout_ref):
    i = pl.program_id(0)
    u0 = i * _UBLK
    urow = lax.broadcasted_iota(jnp.int32, (_UBLK, 1, 1), 0) + u0
    umask = urow == uids_ref[...].reshape(1, 1, -1)
    imask = urow == iids_ref[...].reshape(1, 1, -1)
    uout_ref[...] = jnp.where(umask, newuT_ref[...][None], pu_ref[...])
    iout_ref[...] = jnp.where(imask, newiT_ref[...][None], pi_ref[...])


def _select_scatter(uids, iids, newuT, newiT, pu, pi):
    N, E, B = pu.shape
    grid = (N // _UBLK,)
    const2 = lambda i: (0, 0)
    const3 = lambda i: (i, 0, 0)
    bank_spec = pl.BlockSpec((_UBLK, E, B), const3)
    return pl.pallas_call(
        _select_body,
        grid=grid,
        in_specs=[
            pl.BlockSpec((1, B), const2),
            pl.BlockSpec((1, B), const2),
            pl.BlockSpec((E, B), const2),
            pl.BlockSpec((E, B), const2),
            bank_spec,
            bank_spec,
        ],
        out_specs=(bank_spec, bank_spec),
        out_shape=(jax.ShapeDtypeStruct((N, E, B), pu.dtype),
                   jax.ShapeDtypeStruct((N, E, B), pi.dtype)),
    )(uids.reshape(1, B), iids.reshape(1, B), newuT, newiT, pu, pi)


def kernel(inputs, user_memory, item_memory, uW_ih, uW_hh, ub_ih, ub_hh,
           iW_ih, iW_hh, ib_ih, ib_hh):
    B = inputs.shape[0]
    uid = inputs[:, 0].astype(jnp.int32)
    iid = inputs[:, 1].astype(jnp.int32)
    pu = jnp.transpose(user_memory, (1, 2, 0))
    pi = jnp.transpose(item_memory, (1, 2, 0))

    newuT, newiT, out = _gather_gru(uid, iid, inputs, uW_ih, ub_ih, ub_hh,
                                    iW_ih, ib_ih, ib_hh, pu, pi)
    pu_new, pi_new = _select_scatter(uid, iid, newuT, newiT, pu, pi)
    new_user_memory = jnp.transpose(pu_new, (2, 0, 1))
    new_item_memory = jnp.transpose(pi_new, (2, 0, 1))
    return (out, new_user_memory, new_item_memory)


# speedup vs baseline: 8.2344x; 1.0848x over previous
import jax
import jax.numpy as jnp
from jax import lax
from jax.experimental import pallas as pl
from jax.experimental.pallas import tpu as pltpu

EMBED = 32
UF = 16
IF = 16


_LANES = 128


def _gather_gru_body(uid_ref, iid_ref, inp_ref, uW_ref, ubih_ref, ubhh_ref,
                     iW_ref, ibih_ref, ibhh_ref, pu_ref, pi_ref,
                     newuT_ref, newiT_ref, out_ref,
                     umscr, imscr, semu, semi):
    B = inp_ref.shape[0]

    def issue(b, c):
        u = uid_ref[b]
        it = iid_ref[b]
        lt = pl.multiple_of((b // _LANES) * _LANES, _LANES)
        pltpu.make_async_copy(pu_ref.at[pl.ds(u, 1), :, pl.ds(lt, _LANES)],
                              umscr.at[pl.ds(b, 1)], semu).start()
        pltpu.make_async_copy(pi_ref.at[pl.ds(it, 1), :, pl.ds(lt, _LANES)],
                              imscr.at[pl.ds(b, 1)], semi).start()
        return c

    lax.fori_loop(0, B, issue, 0)
    du = pltpu.make_async_copy(pu_ref.at[pl.ds(0, 1), :, pl.ds(0, _LANES)],
                               umscr.at[pl.ds(0, 1)], semu)
    di = pltpu.make_async_copy(pi_ref.at[pl.ds(0, 1), :, pl.ds(0, _LANES)],
                               imscr.at[pl.ds(0, 1)], semi)

    def drain(b, c):
        du.wait()
        di.wait()
        return c

    lax.fori_loop(0, B, drain, 0)

    def extract(scr):
        s4 = scr[...].reshape(B // _LANES, _LANES, EMBED, _LANES)
        sel = lax.broadcasted_iota(jnp.int32, s4.shape, 1)
        lane = lax.broadcasted_iota(jnp.int32, s4.shape, 3)
        return jnp.sum(jnp.where(sel == lane, s4, 0.0), axis=3).reshape(B, EMBED)

    um = extract(umscr)
    im = extract(imscr)
    inp = inp_ref[...]
    uf = inp[:, 2:2 + UF]
    itf = inp[:, 2 + UF:2 + UF + IF]
    xu = jnp.concatenate([um, uf, im, itf], axis=1)
    xi = jnp.concatenate([im, itf, um, uf], axis=1)

    def cell(x, W, bih, bhh):
        g = lax.dot_general(x, W, (((1,), (1,)), ((), ())),
                            preferred_element_type=jnp.float32)
        g = g + bih
        r = jax.nn.sigmoid(g[:, :EMBED] + bhh[:, :EMBED])
        z = jax.nn.sigmoid(g[:, EMBED:2 * EMBED] + bhh[:, EMBED:2 * EMBED])
        n = jnp.tanh(g[:, 2 * EMBED:] + r * bhh[:, 2 * EMBED:])
        h = (1.0 - z) * n
        norm = jnp.maximum(jnp.sqrt(jnp.sum(h * h, axis=1, keepdims=True)), 1e-12)
        return h / norm

    nu = cell(xu, uW_ref[...], ubih_ref[...], ubhh_ref[...])
    ni = cell(xi, iW_ref[...], ibih_ref[...], ibhh_ref[...])
    newuT_ref[...] = nu.T
    newiT_ref[...] = ni.T
    out_ref[...] = jnp.concatenate([inp[:, :2], nu, ni], axis=1)


def _gather_gru(uid, iid, inputs, uW_ih, ub_ih, ub_hh, iW_ih, ib_ih, ib_hh,
                pu, pi):
    B = inputs.shape[0]
    f32 = jnp.float32
    smem = pl.BlockSpec(memory_space=pltpu.MemorySpace.SMEM)
    hbm = pl.BlockSpec(memory_space=pltpu.MemorySpace.HBM)
    vmem = pl.BlockSpec(memory_space=pltpu.MemorySpace.VMEM)
    return pl.pallas_call(
        _gather_gru_body,
        in_specs=[smem, smem, vmem, vmem, vmem, vmem, vmem, vmem, vmem,
                  hbm, hbm],
        out_shape=(
            jax.ShapeDtypeStruct((EMBED, B), f32),
            jax.ShapeDtypeStruct((EMBED, B), f32),
            jax.ShapeDtypeStruct((B, 2 + 2 * EMBED), f32),
        ),
        scratch_shapes=[
            pltpu.VMEM((B, EMBED, _LANES), f32),
            pltpu.VMEM((B, EMBED, _LANES), f32),
            pltpu.SemaphoreType.DMA,
            pltpu.SemaphoreType.DMA,
        ],
    )(uid, iid, inputs,
      uW_ih, ub_ih.reshape(1, -1), ub_hh.reshape(1, -1),
      iW_ih, ib_ih.reshape(1, -1), ib_hh.reshape(1, -1),
      pu, pi)


_UBLK = 16


def _select_body(uids_ref, iids_ref, newuT_ref, newiT_ref, pu_ref, pi_ref,
                 uout_ref, iout_ref):
    i = pl.program_id(0)
    u0 = i * _UBLK
    urow = lax.broadcasted_iota(jnp.int32, (_UBLK, 1, 1), 0) + u0
    umask = urow == uids_ref[...].reshape(1, 1, -1)
    imask = urow == iids_ref[...].reshape(1, 1, -1)
    uout_ref[...] = jnp.where(umask, newuT_ref[...][None], pu_ref[...])
    iout_ref[...] = jnp.where(imask, newiT_ref[...][None], pi_ref[...])


def _select_scatter(uids, iids, newuT, newiT, pu, pi):
    N, E, B = pu.shape
    grid = (N // _UBLK,)
    const2 = lambda i: (0, 0)
    const3 = lambda i: (i, 0, 0)
    bank_spec = pl.BlockSpec((_UBLK, E, B), const3)
    return pl.pallas_call(
        _select_body,
        grid=grid,
        in_specs=[
            pl.BlockSpec((1, B), const2),
            pl.BlockSpec((1, B), const2),
            pl.BlockSpec((E, B), const2),
            pl.BlockSpec((E, B), const2),
            bank_spec,
            bank_spec,
        ],
        out_specs=(bank_spec, bank_spec),
        out_shape=(jax.ShapeDtypeStruct((N, E, B), pu.dtype),
                   jax.ShapeDtypeStruct((N, E, B), pi.dtype)),
    )(uids.reshape(1, B), iids.reshape(1, B), newuT, newiT, pu, pi)


def kernel(inputs, user_memory, item_memory, uW_ih, uW_hh, ub_ih, ub_hh,
           iW_ih, iW_hh, ib_ih, ib_hh):
    B = inputs.shape[0]
    uid = inputs[:, 0].astype(jnp.int32)
    iid = inputs[:, 1].astype(jnp.int32)
    pu = jnp.transpose(user_memory, (1, 2, 0))
    pi = jnp.transpose(item_memory, (1, 2, 0))

    newuT, newiT, out = _gather_gru(uid, iid, inputs, uW_ih, ub_ih, ub_hh,
                                    iW_ih, ib_ih, ib_hh, pu, pi)
    pu_new, pi_new = _select_scatter(uid, iid, newuT, newiT, pu, pi)
    new_user_memory = jnp.transpose(pu_new, (2, 0, 1))
    new_item_memory = jnp.transpose(pi_new, (2, 0, 1))
    return (out, new_user_memory, new_item_memory)


# speedup vs baseline: 8.2805x; 1.0056x over previous
import jax
import jax.numpy as jnp
from jax import lax
from jax.experimental import pallas as pl
from jax.experimental.pallas import tpu as pltpu

EMBED = 32
UF = 16
IF = 16


_LANES = 128


def _gather_gru_body(uid_ref, iid_ref, inp_ref, uW_ref, ubih_ref, ubhh_ref,
                     iW_ref, ibih_ref, ibhh_ref, pu_ref, pi_ref,
                     newuT_ref, newiT_ref, out_ref,
                     umscr, imscr, semu, semi):
    B = inp_ref.shape[0]

    def issue(b, c):
        u = uid_ref[b]
        it = iid_ref[b]
        lt = pl.multiple_of((b // _LANES) * _LANES, _LANES)
        pltpu.make_async_copy(pu_ref.at[pl.ds(u, 1), :, pl.ds(lt, _LANES)],
                              umscr.at[pl.ds(b, 1)], semu).start()
        pltpu.make_async_copy(pi_ref.at[pl.ds(it, 1), :, pl.ds(lt, _LANES)],
                              imscr.at[pl.ds(b, 1)], semi).start()
        return c

    lax.fori_loop(0, B, issue, 0)
    du = pltpu.make_async_copy(pu_ref.at[pl.ds(0, 1), :, pl.ds(0, _LANES)],
                               umscr.at[pl.ds(0, 1)], semu)
    di = pltpu.make_async_copy(pi_ref.at[pl.ds(0, 1), :, pl.ds(0, _LANES)],
                               imscr.at[pl.ds(0, 1)], semi)

    def drain(b, c):
        du.wait()
        di.wait()
        return c

    lax.fori_loop(0, B, drain, 0)

    def extract(scr):
        s4 = scr[...].reshape(B // _LANES, _LANES, EMBED, _LANES)
        sel = lax.broadcasted_iota(jnp.int32, s4.shape, 1)
        lane = lax.broadcasted_iota(jnp.int32, s4.shape, 3)
        return jnp.sum(jnp.where(sel == lane, s4, 0.0), axis=3).reshape(B, EMBED)

    um = extract(umscr)
    im = extract(imscr)
    inp = inp_ref[...]
    uf = inp[:, 2:2 + UF]
    itf = inp[:, 2 + UF:2 + UF + IF]
    xu = jnp.concatenate([um, uf, im, itf], axis=1)
    xi = jnp.concatenate([im, itf, um, uf], axis=1)

    def cell(x, W, bih, bhh):
        g = lax.dot_general(x, W, (((1,), (1,)), ((), ())),
                            preferred_element_type=jnp.float32)
        g = g + bih
        r = jax.nn.sigmoid(g[:, :EMBED] + bhh[:, :EMBED])
        z = jax.nn.sigmoid(g[:, EMBED:2 * EMBED] + bhh[:, EMBED:2 * EMBED])
        n = jnp.tanh(g[:, 2 * EMBED:] + r * bhh[:, 2 * EMBED:])
        h = (1.0 - z) * n
        norm = jnp.maximum(jnp.sqrt(jnp.sum(h * h, axis=1, keepdims=True)), 1e-12)
        return h / norm

    nu = cell(xu, uW_ref[...], ubih_ref[...], ubhh_ref[...])
    ni = cell(xi, iW_ref[...], ibih_ref[...], ibhh_ref[...])
    newuT_ref[...] = nu.T
    newiT_ref[...] = ni.T
    out_ref[...] = jnp.concatenate([inp[:, :2], nu, ni], axis=1)


def _gather_gru(uid, iid, inputs, uW_ih, ub_ih, ub_hh, iW_ih, ib_ih, ib_hh,
                pu, pi):
    B = inputs.shape[0]
    f32 = jnp.float32
    smem = pl.BlockSpec(memory_space=pltpu.MemorySpace.SMEM)
    hbm = pl.BlockSpec(memory_space=pltpu.MemorySpace.HBM)
    vmem = pl.BlockSpec(memory_space=pltpu.MemorySpace.VMEM)
    return pl.pallas_call(
        _gather_gru_body,
        in_specs=[smem, smem, vmem, vmem, vmem, vmem, vmem, vmem, vmem,
                  hbm, hbm],
        out_shape=(
            jax.ShapeDtypeStruct((EMBED, B), f32),
            jax.ShapeDtypeStruct((EMBED, B), f32),
            jax.ShapeDtypeStruct((B, 2 + 2 * EMBED), f32),
        ),
        scratch_shapes=[
            pltpu.VMEM((B, EMBED, _LANES), f32),
            pltpu.VMEM((B, EMBED, _LANES), f32),
            pltpu.SemaphoreType.DMA,
            pltpu.SemaphoreType.DMA,
        ],
    )(uid, iid, inputs,
      uW_ih, ub_ih.reshape(1, -1), ub_hh.reshape(1, -1),
      iW_ih, ib_ih.reshape(1, -1), ib_hh.reshape(1, -1),
      pu, pi)


_UBLK = 40


def _select_body(uids_ref, iids_ref, newuT_ref, newiT_ref, pu_ref, pi_ref,
                 uout_ref, iout_ref):
    i = pl.program_id(0)
    u0 = i * _UBLK
    urow = lax.broadcasted_iota(jnp.int32, (_UBLK, 1, 1), 0) + u0
    umask = urow == uids_ref[...].reshape(1, 1, -1)
    imask = urow == iids_ref[...].reshape(1, 1, -1)
    uout_ref[...] = jnp.where(umask, newuT_ref[...][None], pu_ref[...])
    iout_ref[...] = jnp.where(imask, newiT_ref[...][None], pi_ref[...])


def _select_scatter(uids, iids, newuT, newiT, pu, pi):
    N, E, B = pu.shape
    grid = (N // _UBLK,)
    const2 = lambda i: (0, 0)
    const3 = lambda i: (i, 0, 0)
    bank_spec = pl.BlockSpec((_UBLK, E, B), const3)
    return pl.pallas_call(
        _select_body,
        grid=grid,
        in_specs=[
            pl.BlockSpec((1, B), const2),
            pl.BlockSpec((1, B), const2),
            pl.BlockSpec((E, B), const2),
            pl.BlockSpec((E, B), const2),
            bank_spec,
            bank_spec,
        ],
        out_specs=(bank_spec, bank_spec),
        out_shape=(jax.ShapeDtypeStruct((N, E, B), pu.dtype),
                   jax.ShapeDtypeStruct((N, E, B), pi.dtype)),
    )(uids.reshape(1, B), iids.reshape(1, B), newuT, newiT, pu, pi)


def kernel(inputs, user_memory, item_memory, uW_ih, uW_hh, ub_ih, ub_hh,
           iW_ih, iW_hh, ib_ih, ib_hh):
    B = inputs.shape[0]
    uid = inputs[:, 0].astype(jnp.int32)
    iid = inputs[:, 1].astype(jnp.int32)
    pu = jnp.transpose(user_memory, (1, 2, 0))
    pi = jnp.transpose(item_memory, (1, 2, 0))

    newuT, newiT, out = _gather_gru(uid, iid, inputs, uW_ih, ub_ih, ub_hh,
                                    iW_ih, ib_ih, ib_hh, pu, pi)
    pu_new, pi_new = _select_scatter(uid, iid, newuT, newiT, pu, pi)
    new_user_memory = jnp.transpose(pu_new, (2, 0, 1))
    new_item_memory = jnp.transpose(pi_new, (2, 0, 1))
    return (out, new_user_memory, new_item_memory)


# speedup vs baseline: 8.7220x; 1.0533x over previous
import jax
import jax.numpy as jnp
from jax import lax
from jax.experimental import pallas as pl
from jax.experimental.pallas import tpu as pltpu

EMBED = 32
UF = 16
IF = 16


_LANES = 128


def _gather_gru_body(uid_ref, iid_ref, inp_ref, uW_ref, ubih_ref, ubhh_ref,
                     iW_ref, ibih_ref, ibhh_ref, pu_ref, pi_ref,
                     newuT_ref, newiT_ref, out_ref,
                     umscr, imscr, semu, semi):
    B = inp_ref.shape[0]

    def issue(b, c):
        u = uid_ref[b]
        it = iid_ref[b]
        lt = pl.multiple_of((b // _LANES) * _LANES, _LANES)
        pltpu.make_async_copy(pu_ref.at[pl.ds(u, 1), :, pl.ds(lt, _LANES)],
                              umscr.at[pl.ds(b, 1)], semu).start()
        pltpu.make_async_copy(pi_ref.at[pl.ds(it, 1), :, pl.ds(lt, _LANES)],
                              imscr.at[pl.ds(b, 1)], semi).start()
        return c

    lax.fori_loop(0, B, issue, 0, unroll=8)
    for t in range(B // _LANES):
        src = pu_ref.at[pl.ds(0, _LANES), :, pl.ds(0, _LANES)]
        pltpu.make_async_copy(src, umscr.at[pl.ds(t * _LANES, _LANES)],
                              semu).wait()
        pltpu.make_async_copy(src, imscr.at[pl.ds(t * _LANES, _LANES)],
                              semi).wait()

    def extract(scr):
        s4 = scr[...].reshape(B // _LANES, _LANES, EMBED, _LANES)
        sel = lax.broadcasted_iota(jnp.int32, s4.shape, 1)
        lane = lax.broadcasted_iota(jnp.int32, s4.shape, 3)
        return jnp.sum(jnp.where(sel == lane, s4, 0.0), axis=3).reshape(B, EMBED)

    um = extract(umscr)
    im = extract(imscr)
    inp = inp_ref[...]
    uf = inp[:, 2:2 + UF]
    itf = inp[:, 2 + UF:2 + UF + IF]
    xu = jnp.concatenate([um, uf, im, itf], axis=1)
    xi = jnp.concatenate([im, itf, um, uf], axis=1)

    def cell(x, W, bih, bhh):
        g = lax.dot_general(x, W, (((1,), (1,)), ((), ())),
                            preferred_element_type=jnp.float32)
        g = g + bih
        r = jax.nn.sigmoid(g[:, :EMBED] + bhh[:, :EMBED])
        z = jax.nn.sigmoid(g[:, EMBED:2 * EMBED] + bhh[:, EMBED:2 * EMBED])
        n = jnp.tanh(g[:, 2 * EMBED:] + r * bhh[:, 2 * EMBED:])
        h = (1.0 - z) * n
        norm = jnp.maximum(jnp.sqrt(jnp.sum(h * h, axis=1, keepdims=True)), 1e-12)
        return h / norm

    nu = cell(xu, uW_ref[...], ubih_ref[...], ubhh_ref[...])
    ni = cell(xi, iW_ref[...], ibih_ref[...], ibhh_ref[...])
    newuT_ref[...] = nu.T
    newiT_ref[...] = ni.T
    out_ref[...] = jnp.concatenate([inp[:, :2], nu, ni], axis=1)


def _gather_gru(uid, iid, inputs, uW_ih, ub_ih, ub_hh, iW_ih, ib_ih, ib_hh,
                pu, pi):
    B = inputs.shape[0]
    f32 = jnp.float32
    smem = pl.BlockSpec(memory_space=pltpu.MemorySpace.SMEM)
    hbm = pl.BlockSpec(memory_space=pltpu.MemorySpace.HBM)
    vmem = pl.BlockSpec(memory_space=pltpu.MemorySpace.VMEM)
    return pl.pallas_call(
        _gather_gru_body,
        in_specs=[smem, smem, vmem, vmem, vmem, vmem, vmem, vmem, vmem,
                  hbm, hbm],
        out_shape=(
            jax.ShapeDtypeStruct((EMBED, B), f32),
            jax.ShapeDtypeStruct((EMBED, B), f32),
            jax.ShapeDtypeStruct((B, 2 + 2 * EMBED), f32),
        ),
        scratch_shapes=[
            pltpu.VMEM((B, EMBED, _LANES), f32),
            pltpu.VMEM((B, EMBED, _LANES), f32),
            pltpu.SemaphoreType.DMA,
            pltpu.SemaphoreType.DMA,
        ],
    )(uid, iid, inputs,
      uW_ih, ub_ih.reshape(1, -1), ub_hh.reshape(1, -1),
      iW_ih, ib_ih.reshape(1, -1), ib_hh.reshape(1, -1),
      pu, pi)


_UBLK = 40


def _select_body(uids_ref, iids_ref, newuT_ref, newiT_ref, pu_ref, pi_ref,
                 uout_ref, iout_ref):
    i = pl.program_id(0)
    u0 = i * _UBLK
    urow = lax.broadcasted_iota(jnp.int32, (_UBLK, 1, 1), 0) + u0
    umask = urow == uids_ref[...].reshape(1, 1, -1)
    imask = urow == iids_ref[...].reshape(1, 1, -1)
    uout_ref[...] = jnp.where(umask, newuT_ref[...][None], pu_ref[...])
    iout_ref[...] = jnp.where(imask, newiT_ref[...][None], pi_ref[...])


def _select_scatter(uids, iids, newuT, newiT, pu, pi):
    N, E, B = pu.shape
    grid = (N // _UBLK,)
    const2 = lambda i: (0, 0)
    const3 = lambda i: (i, 0, 0)
    bank_spec = pl.BlockSpec((_UBLK, E, B), const3)
    return pl.pallas_call(
        _select_body,
        grid=grid,
        in_specs=[
            pl.BlockSpec((1, B), const2),
            pl.BlockSpec((1, B), const2),
            pl.BlockSpec((E, B), const2),
            pl.BlockSpec((E, B), const2),
            bank_spec,
            bank_spec,
        ],
        out_specs=(bank_spec, bank_spec),
        out_shape=(jax.ShapeDtypeStruct((N, E, B), pu.dtype),
                   jax.ShapeDtypeStruct((N, E, B), pi.dtype)),
    )(uids.reshape(1, B), iids.reshape(1, B), newuT, newiT, pu, pi)


def kernel(inputs, user_memory, item_memory, uW_ih, uW_hh, ub_ih, ub_hh,
           iW_ih, iW_hh, ib_ih, ib_hh):
    B = inputs.shape[0]
    uid = inputs[:, 0].astype(jnp.int32)
    iid = inputs[:, 1].astype(jnp.int32)
    pu = jnp.transpose(user_memory, (1, 2, 0))
    pi = jnp.transpose(item_memory, (1, 2, 0))

    newuT, newiT, out = _gather_gru(uid, iid, inputs, uW_ih, ub_ih, ub_hh,
                                    iW_ih, ib_ih, ib_hh, pu, pi)
    pu_new, pi_new = _select_scatter(uid, iid, newuT, newiT, pu, pi)
    new_user_memory = jnp.transpose(pu_new, (2, 0, 1))
    new_item_memory = jnp.transpose(pi_new, (2, 0, 1))
    return (out, new_user_memory, new_item_memory)
